# baseline (speedup 1.0000x reference)
import functools

import jax
import jax.numpy as jnp
from jax import lax
from jax.experimental import pallas as pl
from jax.experimental.pallas import tpu as pltpu
from jax.experimental.pallas import tpu_sc as plsc

B, C, H, W = 32, 2, 512, 512
P = 128
L = 16
NC, NS = 2, 16
NW = NC * NS

_CH = 1
_BATCH_STRIDE = C * H * W
_CH_OFF = _CH * H * W


def _make_sc_gather():
    mesh = plsc.VectorSubcoreMesh(core_axis_name="c", subcore_axis_name="s")

    @functools.partial(
        pl.kernel,
        mesh=mesh,
        out_type=jax.ShapeDtypeStruct((B * P,), jnp.float32),
        scratch_types=[
            pltpu.VMEM((P,), jnp.int32),
            pltpu.VMEM((P,), jnp.int32),
            pltpu.VMEM((P,), jnp.int32),
            pltpu.VMEM((P,), jnp.float32),
            pltpu.SemaphoreType.DMA,
        ],
    )
    def k(x_hbm, px_hbm, py_hbm, out_hbm, px_v, py_v, idx_v, val_v, sem):
        wid = lax.axis_index("s") * NC + lax.axis_index("c")
        pltpu.sync_copy(px_hbm, px_v)
        pltpu.sync_copy(py_hbm, py_v)
        base = wid * _BATCH_STRIDE + _CH_OFF
        for j in range(P // L):
            sl = pl.ds(j * L, L)
            idx_v[sl] = px_v[sl] * W + py_v[sl] + base
        pltpu.async_copy(x_hbm.at[idx_v], val_v, sem).wait()
        pltpu.sync_copy(val_v, out_hbm.at[pl.ds(wid * P, P)])

    return k


_sc_gather = _make_sc_gather()


def kernel(x, probe_x, probe_y):
    flat = _sc_gather(x.reshape(-1), probe_x, probe_y)
    return flat.reshape(B, P)


# speedup vs baseline: 2.8351x; 2.8351x over previous
import functools

import jax
import jax.numpy as jnp
from jax import lax
from jax.experimental import pallas as pl
from jax.experimental.pallas import tpu as pltpu
from jax.experimental.pallas import tpu_sc as plsc

B, C, H, W = 32, 2, 512, 512
P = 128
L = 16
NC, NS = 2, 16
NW = NC * NS

_CH = 1
_ROWS = B * C * H


def _make_sc_gather():
    mesh = plsc.VectorSubcoreMesh(core_axis_name="c", subcore_axis_name="s")

    @functools.partial(
        pl.kernel,
        mesh=mesh,
        out_type=jax.ShapeDtypeStruct((B * P,), jnp.float32),
        compiler_params=pltpu.CompilerParams(needs_layout_passes=False),
        scratch_types=[
            pltpu.VMEM((P,), jnp.int32),
            pltpu.VMEM((P,), jnp.int32),
            pltpu.VMEM((P,), jnp.int32),
            pltpu.VMEM((P, W), jnp.float32),
            pltpu.VMEM((P,), jnp.float32),
            pltpu.SemaphoreType.DMA,
        ],
    )
    def k(x_hbm, px_hbm, py_hbm, out_hbm, px_v, py_v, rid_v, rows_v, val_v, sem):
        wid = lax.axis_index("s") * NC + lax.axis_index("c")
        pltpu.sync_copy(px_hbm, px_v)
        pltpu.sync_copy(py_hbm, py_v)
        base = (wid * C + _CH) * H
        for j in range(P // L):
            sl = pl.ds(j * L, L)
            rid_v[sl] = px_v[sl] + base
        pltpu.async_copy(x_hbm.at[rid_v], rows_v, sem).wait()
        for j in range(P // L):
            sl = pl.ds(j * L, L)
            rows = lax.iota(jnp.int32, L) + j * L
            val_v[sl] = plsc.load_gather(rows_v, [rows, py_v[sl]])
        pltpu.sync_copy(val_v, out_hbm.at[pl.ds(wid * P, P)])

    return k


_sc_gather = _make_sc_gather()


def kernel(x, probe_x, probe_y):
    flat = _sc_gather(x.reshape(_ROWS, W), probe_x, probe_y)
    return flat.reshape(B, P)


# speedup vs baseline: 2.8441x; 1.0032x over previous
import functools

import jax
import jax.numpy as jnp
from jax import lax
from jax.experimental import pallas as pl
from jax.experimental.pallas import tpu as pltpu
from jax.experimental.pallas import tpu_sc as plsc

B, C, H, W = 32, 2, 512, 512
P = 128
L = 16
NC, NS = 2, 16
NW = NC * NS

_CH = 1
_ROWS = B * C * H


def _make_sc_gather():
    mesh = plsc.VectorSubcoreMesh(core_axis_name="c", subcore_axis_name="s")

    @functools.partial(
        pl.kernel,
        mesh=mesh,
        out_type=jax.ShapeDtypeStruct((B, P), jnp.float32),
        compiler_params=pltpu.CompilerParams(
            needs_layout_passes=False,
            skip_device_barrier=True,
            disable_bounds_checks=True,
            disable_semaphore_checks=True,
        ),
        scratch_types=[
            pltpu.VMEM((P,), jnp.int32),
            pltpu.VMEM((P,), jnp.int32),
            pltpu.VMEM((P,), jnp.int32),
            pltpu.VMEM((P, W), jnp.float32),
            pltpu.VMEM((P,), jnp.float32),
            pltpu.SemaphoreType.DMA,
        ],
    )
    def k(x_hbm, px_hbm, py_hbm, out_hbm, px_v, py_v, rid_v, rows_v, val_v, sem):
        wid = lax.axis_index("s") * NC + lax.axis_index("c")
        pltpu.sync_copy(px_hbm, px_v)
        pltpu.sync_copy(py_hbm, py_v)
        base = (wid * C + _CH) * H
        for j in range(P // L):
            sl = pl.ds(j * L, L)
            rid_v[sl] = px_v[sl] + base
        pltpu.async_copy(x_hbm.at[rid_v], rows_v, sem).wait()
        for j in range(P // L):
            sl = pl.ds(j * L, L)
            rows = lax.iota(jnp.int32, L) + j * L
            val_v[sl] = plsc.load_gather(rows_v, [rows, py_v[sl]])
        pltpu.sync_copy(val_v, out_hbm.at[wid])

    return k


_sc_gather = _make_sc_gather()


def kernel(x, probe_x, probe_y):
    return _sc_gather(x.reshape(_ROWS, W), probe_x, probe_y)


# speedup vs baseline: 2.8681x; 1.0084x over previous
import functools

import jax
import jax.numpy as jnp
from jax import lax
from jax.experimental import pallas as pl
from jax.experimental.pallas import tpu as pltpu
from jax.experimental.pallas import tpu_sc as plsc

B, C, H, W = 32, 2, 512, 512
P = 128
L = 16
NC, NS = 2, 16
NW = NC * NS

_CH = 1
_ROWS = B * C * H


def _make_sc_gather():
    mesh = plsc.VectorSubcoreMesh(core_axis_name="c", subcore_axis_name="s")

    @functools.partial(
        pl.kernel,
        mesh=mesh,
        out_type=jax.ShapeDtypeStruct((B, P), jnp.float32),
        compiler_params=pltpu.CompilerParams(
            needs_layout_passes=False,
            skip_device_barrier=True,
            disable_bounds_checks=True,
            disable_semaphore_checks=True,
        ),
        scratch_types=[
            pltpu.VMEM((P,), jnp.int32),
            pltpu.VMEM((P,), jnp.int32),
            pltpu.VMEM((P // 2,), jnp.int32),
            pltpu.VMEM((P // 2,), jnp.int32),
            pltpu.VMEM((P // 2, W), jnp.float32),
            pltpu.VMEM((P // 2, W), jnp.float32),
            pltpu.VMEM((P,), jnp.float32),
            pltpu.SemaphoreType.DMA,
            pltpu.SemaphoreType.DMA,
            pltpu.SemaphoreType.DMA,
        ],
    )
    def k(x_hbm, px_hbm, py_hbm, out_hbm,
          px_v, py_v, rid0, rid1, rows0, rows1, val_v, sem0, sem1, sem2):
        wid = lax.axis_index("s") * NC + lax.axis_index("c")
        half = P // 2
        cpx = pltpu.async_copy(px_hbm, px_v, sem0)
        cpy = pltpu.async_copy(py_hbm, py_v, sem1)
        cpx.wait()
        base = (wid * C + _CH) * H
        for j in range(half // L):
            sl = pl.ds(j * L, L)
            rid0[sl] = px_v[sl] + base
        g0 = pltpu.async_copy(x_hbm.at[rid0], rows0, sem0)
        for j in range(half // L):
            sl = pl.ds(j * L, L)
            rid1[sl] = px_v[pl.ds(half + j * L, L)] + base
        g1 = pltpu.async_copy(x_hbm.at[rid1], rows1, sem2)
        cpy.wait()
        g0.wait()
        for j in range(half // L):
            rows = lax.iota(jnp.int32, L) + j * L
            val_v[pl.ds(j * L, L)] = plsc.load_gather(
                rows0, [rows, py_v[pl.ds(j * L, L)]])
        g1.wait()
        for j in range(half // L):
            rows = lax.iota(jnp.int32, L) + j * L
            val_v[pl.ds(half + j * L, L)] = plsc.load_gather(
                rows1, [rows, py_v[pl.ds(half + j * L, L)]])
        pltpu.sync_copy(val_v, out_hbm.at[wid])

    return k


_sc_gather = _make_sc_gather()


def kernel(x, probe_x, probe_y):
    return _sc_gather(x.reshape(_ROWS, W), probe_x, probe_y)
